# baseline (speedup 1.0000x reference)
import functools

import jax
import jax.numpy as jnp
from jax import lax
from jax.experimental import pallas as pl
from jax.experimental.pallas import tpu as pltpu
from jax.experimental.pallas import tpu_sc as plsc

N = 10000
D = 128
E = 320000

NC = 2
NS = 16
CHUNK = 128
NPAD = 10240
EPAD = 327680
ROWS = EPAD // CHUNK
ROWS_PER_SC = ROWS // NC
ROWS_PER_TILE = ROWS_PER_SC // NS
NSLICE = NPAD // NS

_mesh = plsc.VectorSubcoreMesh(core_axis_name="c", subcore_axis_name="s")


_HIST_DEPTH = 8


@functools.partial(
    pl.kernel,
    out_type=jax.ShapeDtypeStruct((NC, 2, NPAD), jnp.float32),
    mesh=_mesh,
    scratch_types=[
        pltpu.VMEM((2 * ROWS_PER_TILE, CHUNK), jnp.int32),
        pltpu.VMEM((CHUNK,), jnp.float32),
        pltpu.VMEM((NSLICE,), jnp.float32),
        pltpu.VMEM_SHARED((2 * NPAD,), jnp.float32),
        pltpu.SemaphoreType.DMA,
    ],
)
def _hist_kernel(src_hbm, dst2_hbm, hist_hbm, idx_v, ones_v, zslice_v,
                 hist_s, sem):
    cid = lax.axis_index("c")
    sid = lax.axis_index("s")

    for j in range(CHUNK // 16):
        ones_v[pl.ds(j * 16, 16)] = jnp.ones((16,), jnp.float32)

    def _zfill(i, carry):
        zslice_v[pl.ds(i * 16, 16)] = jnp.zeros((16,), jnp.float32)
        return carry

    lax.fori_loop(0, NSLICE // 16, _zfill, 0)

    nbase = sid * NSLICE
    pltpu.sync_copy(zslice_v, hist_s.at[pl.ds(nbase, NSLICE)])
    pltpu.sync_copy(zslice_v, hist_s.at[pl.ds(NPAD + nbase, NSLICE)])
    plsc.subcore_barrier()

    row0 = cid * ROWS_PER_SC + sid * ROWS_PER_TILE
    pltpu.sync_copy(src_hbm.at[pl.ds(row0, ROWS_PER_TILE)],
                    idx_v.at[pl.ds(0, ROWS_PER_TILE)])
    pltpu.sync_copy(dst2_hbm.at[pl.ds(row0, ROWS_PER_TILE)],
                    idx_v.at[pl.ds(ROWS_PER_TILE, ROWS_PER_TILE)])

    def _scat_group(g, carry):
        base = g * _HIST_DEPTH
        for j in range(_HIST_DEPTH):
            pltpu.async_copy(ones_v, hist_s.at[idx_v.at[base + j]], sem,
                             add=True)
        for j in range(_HIST_DEPTH):
            pltpu.make_async_copy(ones_v, hist_s.at[idx_v.at[base + j]],
                                  sem).wait()
        return carry

    lax.fori_loop(0, (2 * ROWS_PER_TILE) // _HIST_DEPTH, _scat_group, 0)

    plsc.subcore_barrier()
    pltpu.sync_copy(hist_s.at[pl.ds(nbase, NSLICE)],
                    hist_hbm.at[cid, 0, pl.ds(nbase, NSLICE)])
    pltpu.sync_copy(hist_s.at[pl.ds(NPAD + nbase, NSLICE)],
                    hist_hbm.at[cid, 1, pl.ds(nbase, NSLICE)])


def _prep_body(feat_ref, hist_ref, fs_ref, nin_ref):
    out_deg = hist_ref[0, 0, :] + hist_ref[1, 0, :]
    in_deg = hist_ref[0, 1, :] + hist_ref[1, 1, :]
    norm_out = lax.rsqrt(jnp.maximum(out_deg, 1.0) + 1.0)
    nin_ref[...] = lax.rsqrt(jnp.maximum(in_deg, 1.0) + 1.0)
    fs_ref[...] = feat_ref[...] * norm_out[:, None]


_prep = pl.pallas_call(
    _prep_body,
    out_shape=(
        jax.ShapeDtypeStruct((NPAD, D), jnp.float32),
        jax.ShapeDtypeStruct((NPAD,), jnp.float32),
    ),
)


@functools.partial(
    pl.kernel,
    out_type=jax.ShapeDtypeStruct((NC, NPAD, D), jnp.float32),
    mesh=_mesh,
    scratch_types=[
        pltpu.VMEM((ROWS_PER_TILE // 2, CHUNK), jnp.int32),
        pltpu.VMEM((ROWS_PER_TILE // 2, CHUNK), jnp.int32),
        pltpu.VMEM((CHUNK, D), jnp.float32),
        pltpu.VMEM((CHUNK, D), jnp.float32),
        pltpu.VMEM_SHARED((NPAD, D), jnp.float32),
        pltpu.SemaphoreType.DMA,
        pltpu.SemaphoreType.DMA,
        pltpu.SemaphoreType.DMA,
        pltpu.SemaphoreType.DMA,
    ],
)
def _agg_kernel(featsrc_hbm, src_hbm, dst_hbm, zeros_hbm, part_hbm,
                sidx_v, didx_v, rows0_v, rows1_v, h_s,
                gsem0, gsem1, ssem0, ssem1):
    cid = lax.axis_index("c")
    sid = lax.axis_index("s")
    half_rows = ROWS_PER_TILE // 2

    nbase = sid * NSLICE
    pltpu.sync_copy(zeros_hbm, h_s.at[pl.ds(nbase, NSLICE)])
    plsc.subcore_barrier()

    row0 = cid * ROWS_PER_SC + sid * ROWS_PER_TILE

    for half in range(2):
        r0 = row0 + half * half_rows
        pltpu.sync_copy(src_hbm.at[pl.ds(r0, half_rows)], sidx_v)
        pltpu.sync_copy(dst_hbm.at[pl.ds(r0, half_rows)], didx_v)

        pltpu.async_copy(featsrc_hbm.at[sidx_v.at[0]], rows0_v, gsem0)
        pltpu.async_copy(featsrc_hbm.at[sidx_v.at[1]], rows1_v, gsem1)

        def _edge_pair(t, carry):
            j0 = 2 * t
            pltpu.make_async_copy(featsrc_hbm.at[sidx_v.at[j0]], rows0_v,
                                  gsem0).wait()
            pltpu.async_copy(rows0_v, h_s.at[didx_v.at[j0]], ssem0, add=True)
            pltpu.make_async_copy(featsrc_hbm.at[sidx_v.at[j0 + 1]], rows1_v,
                                  gsem1).wait()
            pltpu.async_copy(rows1_v, h_s.at[didx_v.at[j0 + 1]], ssem1,
                             add=True)

            @pl.when(t < half_rows // 2 - 1)
            def _():
                pltpu.make_async_copy(rows0_v, h_s.at[didx_v.at[j0]],
                                      ssem0).wait()
                pltpu.async_copy(featsrc_hbm.at[sidx_v.at[j0 + 2]], rows0_v,
                                 gsem0)
                pltpu.make_async_copy(rows1_v, h_s.at[didx_v.at[j0 + 1]],
                                      ssem1).wait()
                pltpu.async_copy(featsrc_hbm.at[sidx_v.at[j0 + 3]], rows1_v,
                                 gsem1)

            return carry

        lax.fori_loop(0, half_rows // 2, _edge_pair, 0)

        pltpu.make_async_copy(rows0_v, h_s.at[didx_v.at[half_rows - 2]],
                              ssem0).wait()
        pltpu.make_async_copy(rows1_v, h_s.at[didx_v.at[half_rows - 1]],
                              ssem1).wait()

    plsc.subcore_barrier()
    pltpu.sync_copy(h_s.at[pl.ds(nbase, NSLICE)],
                    part_hbm.at[cid, pl.ds(nbase, NSLICE)])


def _final_body(feat_ref, part_ref, nin_ref, out_ref):
    h = part_ref[0, :N, :] + part_ref[1, :N, :]
    out_ref[...] = (feat_ref[...] + h) * nin_ref[:N][:, None]


_final = pl.pallas_call(
    _final_body,
    out_shape=jax.ShapeDtypeStruct((N, D), jnp.float32),
)


def kernel(feat, edge_index):
    ei = edge_index.astype(jnp.int32)
    pad = jnp.full((EPAD - E,), N, jnp.int32)
    src = jnp.concatenate([ei[0], pad]).reshape(ROWS, CHUNK)
    dst = jnp.concatenate([ei[1], pad]).reshape(ROWS, CHUNK)
    feat_pad = jnp.pad(feat, ((0, NPAD - N), (0, 0)))

    hist = _hist_kernel(src, dst + NPAD)
    feat_src, norm_in = _prep(feat_pad, hist)
    zeros = jnp.zeros((NSLICE, D), jnp.float32)
    part = _agg_kernel(feat_src, src, dst, zeros)
    return _final(feat, part, norm_in)


# speedup vs baseline: 2.7496x; 2.7496x over previous
import functools

import jax
import jax.numpy as jnp
from jax import lax
from jax.experimental import pallas as pl
from jax.experimental.pallas import tpu as pltpu
from jax.experimental.pallas import tpu_sc as plsc

N = 10000
D = 128
E = 320000

NC = 2
NS = 16
CHUNK = 128
NPAD = 10240
EPAD = 327680
ROWS = EPAD // CHUNK
ROWS_PER_SC = ROWS // NC
ROWS_PER_TILE = ROWS_PER_SC // NS
NSLICE = NPAD // NS

_mesh = plsc.VectorSubcoreMesh(core_axis_name="c", subcore_axis_name="s")


_HIST_DEPTH = 8


@functools.partial(
    pl.kernel,
    out_type=jax.ShapeDtypeStruct((NC, 2, NPAD), jnp.float32),
    mesh=_mesh,
    scratch_types=[
        pltpu.VMEM((2 * ROWS_PER_TILE, CHUNK), jnp.int32),
        pltpu.VMEM((CHUNK,), jnp.float32),
        pltpu.VMEM((NSLICE,), jnp.float32),
        pltpu.VMEM_SHARED((2 * NPAD,), jnp.float32),
        pltpu.SemaphoreType.DMA,
    ],
)
def _hist_kernel(src_hbm, dst2_hbm, hist_hbm, idx_v, ones_v, zslice_v,
                 hist_s, sem):
    cid = lax.axis_index("c")
    sid = lax.axis_index("s")

    for j in range(CHUNK // 16):
        ones_v[pl.ds(j * 16, 16)] = jnp.ones((16,), jnp.float32)

    def _zfill(i, carry):
        zslice_v[pl.ds(i * 16, 16)] = jnp.zeros((16,), jnp.float32)
        return carry

    lax.fori_loop(0, NSLICE // 16, _zfill, 0)

    nbase = sid * NSLICE
    pltpu.sync_copy(zslice_v, hist_s.at[pl.ds(nbase, NSLICE)])
    pltpu.sync_copy(zslice_v, hist_s.at[pl.ds(NPAD + nbase, NSLICE)])
    plsc.subcore_barrier()

    row0 = cid * ROWS_PER_SC + sid * ROWS_PER_TILE
    pltpu.sync_copy(src_hbm.at[pl.ds(row0, ROWS_PER_TILE)],
                    idx_v.at[pl.ds(0, ROWS_PER_TILE)])
    pltpu.sync_copy(dst2_hbm.at[pl.ds(row0, ROWS_PER_TILE)],
                    idx_v.at[pl.ds(ROWS_PER_TILE, ROWS_PER_TILE)])

    def _scat_group(g, carry):
        base = g * _HIST_DEPTH
        for j in range(_HIST_DEPTH):
            pltpu.async_copy(ones_v, hist_s.at[idx_v.at[base + j]], sem,
                             add=True)
        for j in range(_HIST_DEPTH):
            pltpu.make_async_copy(ones_v, hist_s.at[idx_v.at[base + j]],
                                  sem).wait()
        return carry

    lax.fori_loop(0, (2 * ROWS_PER_TILE) // _HIST_DEPTH, _scat_group, 0)

    plsc.subcore_barrier()
    pltpu.sync_copy(hist_s.at[pl.ds(nbase, NSLICE)],
                    hist_hbm.at[cid, 0, pl.ds(nbase, NSLICE)])
    pltpu.sync_copy(hist_s.at[pl.ds(NPAD + nbase, NSLICE)],
                    hist_hbm.at[cid, 1, pl.ds(nbase, NSLICE)])


def _prep_body(feat_ref, hist_ref, fs_ref, nin_ref):
    out_deg = hist_ref[0, 0, :] + hist_ref[1, 0, :]
    in_deg = hist_ref[0, 1, :] + hist_ref[1, 1, :]
    norm_out = lax.rsqrt(jnp.maximum(out_deg, 1.0) + 1.0)
    nin_ref[...] = lax.rsqrt(jnp.maximum(in_deg, 1.0) + 1.0)
    fs_ref[...] = feat_ref[...] * norm_out[:, None]


_prep = pl.pallas_call(
    _prep_body,
    out_shape=(
        jax.ShapeDtypeStruct((NPAD, D), jnp.float32),
        jax.ShapeDtypeStruct((NPAD,), jnp.float32),
    ),
)


@functools.partial(
    pl.kernel,
    out_type=jax.ShapeDtypeStruct((NC, NPAD, D), jnp.float32),
    mesh=_mesh,
    scratch_types=[
        pltpu.VMEM((ROWS_PER_TILE // 2, CHUNK), jnp.int32),
        pltpu.VMEM((ROWS_PER_TILE // 2, CHUNK), jnp.int32),
        pltpu.VMEM((CHUNK, D), jnp.float32),
        pltpu.VMEM((CHUNK, D), jnp.float32),
        pltpu.VMEM_SHARED((NPAD, D), jnp.float32),
        pltpu.SemaphoreType.DMA,
        pltpu.SemaphoreType.DMA,
    ],
)
def _agg_kernel(featsrc_hbm, src_hbm, dst_hbm, zeros_hbm, part_hbm,
                sidx_v, didx_v, rows0_v, rows1_v, h_s, sem0, sem1):
    cid = lax.axis_index("c")
    sid = lax.axis_index("s")
    half_rows = ROWS_PER_TILE // 2

    nbase = sid * NSLICE
    pltpu.sync_copy(zeros_hbm, h_s.at[pl.ds(nbase, NSLICE)])
    plsc.subcore_barrier()

    row0 = cid * ROWS_PER_SC + sid * ROWS_PER_TILE

    for half in range(2):
        r0 = row0 + half * half_rows
        pltpu.sync_copy(src_hbm.at[pl.ds(r0, half_rows)], sidx_v)
        pltpu.sync_copy(dst_hbm.at[pl.ds(r0, half_rows)], didx_v)

        pltpu.async_copy(featsrc_hbm.at[sidx_v.at[0]], rows0_v, sem0)

        def _edge_pair(t, carry):
            j0 = 2 * t
            pltpu.async_copy(featsrc_hbm.at[sidx_v.at[j0 + 1]], rows1_v, sem1)
            pltpu.make_async_copy(featsrc_hbm.at[sidx_v.at[j0]], rows0_v,
                                  sem0).wait()
            pltpu.sync_copy(rows0_v, h_s.at[didx_v.at[j0]], add=True)

            @pl.when(t < half_rows // 2 - 1)
            def _():
                pltpu.async_copy(featsrc_hbm.at[sidx_v.at[j0 + 2]], rows0_v,
                                 sem0)

            pltpu.make_async_copy(featsrc_hbm.at[sidx_v.at[j0 + 1]], rows1_v,
                                  sem1).wait()
            pltpu.sync_copy(rows1_v, h_s.at[didx_v.at[j0 + 1]], add=True)
            return carry

        lax.fori_loop(0, half_rows // 2, _edge_pair, 0)

    plsc.subcore_barrier()
    pltpu.sync_copy(h_s.at[pl.ds(nbase, NSLICE)],
                    part_hbm.at[cid, pl.ds(nbase, NSLICE)])


def _final_body(feat_ref, part_ref, nin_ref, out_ref):
    h = part_ref[0, :N, :] + part_ref[1, :N, :]
    out_ref[...] = (feat_ref[...] + h) * nin_ref[:N][:, None]


_final = pl.pallas_call(
    _final_body,
    out_shape=jax.ShapeDtypeStruct((N, D), jnp.float32),
)


def kernel(feat, edge_index):
    ei = edge_index.astype(jnp.int32)
    pad = N + jnp.arange(EPAD - E, dtype=jnp.int32) % (NPAD - N)
    src = jnp.concatenate([ei[0], pad]).reshape(ROWS, CHUNK)
    dst = jnp.concatenate([ei[1], pad]).reshape(ROWS, CHUNK)
    feat_pad = jnp.pad(feat, ((0, NPAD - N), (0, 0)))

    hist = _hist_kernel(src, dst + NPAD)
    feat_src, norm_in = _prep(feat_pad, hist)
    zeros = jnp.zeros((NSLICE, D), jnp.float32)
    part = _agg_kernel(feat_src, src, dst, zeros)
    return _final(feat, part, norm_in)
